# Initial kernel scaffold; baseline (speedup 1.0000x reference)
import jax
import jax.numpy as jnp
from jax.experimental import pallas as pl


def kernel(x, table, W_proj, b_proj):
    raise NotImplementedError("write your pallas kernel here")


# speedup vs baseline: 2.0677x; 2.0677x over previous
import functools

import jax
import jax.numpy as jnp
from jax import lax
from jax.experimental import pallas as pl
from jax.experimental.pallas import tpu as pltpu
from jax.experimental.pallas import tpu_sc as plsc

_NCORES = 2
_NSUB = 16
_NW = _NCORES * _NSUB
_CSZ = 128
_K = 10


def _sc_gather(idx, table):
    nw, chunks, csz = idx.shape
    _, d = table.shape
    per_w = chunks * csz
    n = nw * per_w
    groups = chunks // _K
    half = groups // 2
    mesh = plsc.VectorSubcoreMesh(core_axis_name="c", subcore_axis_name="s")

    @functools.partial(
        pl.kernel,
        out_type=jax.ShapeDtypeStruct((n, d), jnp.float32),
        mesh=mesh,
        compiler_params=pltpu.CompilerParams(use_tc_tiling_on_sc=False),
        scratch_types=[
            pltpu.VMEM((chunks, csz), jnp.int32),
            pltpu.VMEM((2 * _K, csz, d), jnp.float32),
            pltpu.SemaphoreType.DMA,
            pltpu.SemaphoreType.DMA,
        ],
    )
    def gather_kernel(idx_hbm, table_hbm, out_hbm, idx_v, rows_v, gsem, wsem):
        wid = lax.axis_index("s") * _NCORES + lax.axis_index("c")
        base = wid * per_w
        pltpu.sync_copy(idx_hbm.at[wid], idx_v)

        def issue_gathers(g, setoff):
            for b in range(_K):
                pltpu.async_copy(
                    table_hbm.at[idx_v.at[g * _K + b]],
                    rows_v.at[setoff + b],
                    gsem,
                )

        def drain_g(setoff):
            for b in range(_K):
                pltpu.make_async_copy(
                    table_hbm.at[pl.ds(0, csz)], rows_v.at[setoff + b], gsem
                ).wait()

        def issue_wb(g, setoff):
            for b in range(_K):
                pltpu.async_copy(
                    rows_v.at[setoff + b],
                    out_hbm.at[pl.ds(base + (g * _K + b) * csz, csz)],
                    wsem,
                )

        def drain_wb(setoff):
            for b in range(_K):
                pltpu.make_async_copy(
                    rows_v.at[setoff + b], out_hbm.at[pl.ds(0, csz)], wsem
                ).wait()

        issue_gathers(0, 0)

        def body(h, carry):
            ge = 2 * h
            go = 2 * h + 1
            drain_g(0)
            issue_wb(ge, 0)

            @pl.when(h >= 1)
            def _():
                drain_wb(_K)

            issue_gathers(go, _K)
            drain_wb(0)
            @pl.when(h + 1 < half)
            def _():
                issue_gathers(ge + 2, 0)

            drain_g(_K)
            issue_wb(go, _K)
            return carry

        lax.fori_loop(0, half, body, 0)
        drain_wb(_K)

    return gather_kernel(idx, table)


def _tc_head(flat, w, b):
    bsz, f = flat.shape
    nc = w.shape[1]
    bb = 128

    def body(x_ref, w_ref, b_ref, o_ref):
        g = jax.nn.gelu(x_ref[...])
        o_ref[...] = (
            jnp.dot(g, w_ref[...], preferred_element_type=jnp.float32) + b_ref[...]
        )

    return pl.pallas_call(
        body,
        grid=(bsz // bb,),
        in_specs=[
            pl.BlockSpec((bb, f), lambda i: (i, 0)),
            pl.BlockSpec((f, nc), lambda i: (0, 0)),
            pl.BlockSpec((1, nc), lambda i: (0, 0)),
        ],
        out_specs=pl.BlockSpec((bb, nc), lambda i: (i, 0)),
        out_shape=jax.ShapeDtypeStruct((bsz, nc), jnp.float32),
    )(flat, w, b)


def kernel(x, table, W_proj, b_proj):
    bsz, s = x.shape
    _, d = table.shape
    nc = W_proj.shape[1]
    n = bsz * s
    chunks = n // (_NW * _CSZ)
    idx = x.reshape(_NW, chunks, _CSZ).astype(jnp.int32)
    xe = _sc_gather(idx, table)
    flat = xe.reshape(bsz, s * d)
    return _tc_head(flat, W_proj, b_proj.reshape(1, nc))


# speedup vs baseline: 2.1534x; 1.0414x over previous
import functools

import jax
import jax.numpy as jnp
from jax import lax
from jax.experimental import pallas as pl
from jax.experimental.pallas import tpu as pltpu
from jax.experimental.pallas import tpu_sc as plsc

_NCORES = 2
_NSUB = 16
_NW = _NCORES * _NSUB
_CSZ = 128
_K = 10


def _sc_gather(idx, table):
    nw, chunks, csz = idx.shape
    _, d = table.shape
    per_w = chunks * csz
    n = nw * per_w
    groups = chunks // _K
    half = groups // 2
    mesh = plsc.VectorSubcoreMesh(core_axis_name="c", subcore_axis_name="s")

    @functools.partial(
        pl.kernel,
        out_type=jax.ShapeDtypeStruct((n, d), jnp.float32),
        mesh=mesh,
        compiler_params=pltpu.CompilerParams(use_tc_tiling_on_sc=False),
        scratch_types=[
            pltpu.VMEM((chunks, csz), jnp.int32),
            pltpu.VMEM((2 * _K, csz, d), jnp.float32),
            pltpu.SemaphoreType.DMA,
            pltpu.SemaphoreType.DMA,
        ],
    )
    def gather_kernel(idx_hbm, table_hbm, out_hbm, idx_v, rows_v, gsem, wsem):
        wid = lax.axis_index("s") * _NCORES + lax.axis_index("c")
        base = wid * per_w
        pltpu.sync_copy(idx_hbm.at[wid], idx_v)

        def issue_gathers(g, setoff):
            for b in range(_K):
                pltpu.async_copy(
                    table_hbm.at[idx_v.at[g * _K + b]],
                    rows_v.at[setoff + b],
                    gsem,
                )

        def drain_g(setoff):
            for b in range(_K):
                pltpu.make_async_copy(
                    table_hbm.at[pl.ds(0, csz)], rows_v.at[setoff + b], gsem
                ).wait()

        def issue_wb(g, setoff):
            for b in range(_K):
                pltpu.async_copy(
                    rows_v.at[setoff + b],
                    out_hbm.at[pl.ds(base + (g * _K + b) * csz, csz)],
                    wsem,
                )

        def drain_wb(setoff):
            for b in range(_K):
                pltpu.make_async_copy(
                    rows_v.at[setoff + b], out_hbm.at[pl.ds(0, csz)], wsem
                ).wait()

        issue_gathers(0, 0)

        def body(h, carry):
            ge = 2 * h
            go = 2 * h + 1
            drain_g(0)
            issue_wb(ge, 0)

            @pl.when(h >= 1)
            def _():
                drain_wb(_K)

            issue_gathers(go, _K)
            drain_wb(0)
            @pl.when(h + 1 < half)
            def _():
                issue_gathers(ge + 2, 0)

            drain_g(_K)
            issue_wb(go, _K)
            return carry

        lax.fori_loop(0, half, body, 0)
        drain_wb(_K)

    return gather_kernel(idx, table)


def _tc_head(x128, w50, b, bsz, nt):
    nc = w50.shape[2]
    bb = 2048
    nb = bsz // bb

    def body(x_ref, w_ref, b_ref, o_ref):
        t = pl.program_id(1)
        g = jax.nn.gelu(x_ref[...])
        p = jnp.dot(g, w_ref[0], preferred_element_type=jnp.float32)

        @pl.when(t == 0)
        def _():
            o_ref[...] = p + b_ref[...]

        @pl.when(t > 0)
        def _():
            o_ref[...] += p

    return pl.pallas_call(
        body,
        grid=(nb, nt),
        in_specs=[
            pl.BlockSpec((bb, 128), lambda i, t: (t * nb + i, 0)),
            pl.BlockSpec((1, 128, nc), lambda i, t: (t, 0, 0)),
            pl.BlockSpec((1, nc), lambda i, t: (0, 0)),
        ],
        out_specs=pl.BlockSpec((bb, nc), lambda i, t: (i, 0)),
        out_shape=jax.ShapeDtypeStruct((bsz, nc), jnp.float32),
    )(x128, w50, b)


def kernel(x, table, W_proj, b_proj):
    bsz, s = x.shape
    _, d = table.shape
    nc = W_proj.shape[1]
    n = bsz * s
    upack = 128 // d
    nt = s // upack
    chunks = n // (_NW * _CSZ)
    xp = x.reshape(bsz, nt, upack).transpose(1, 0, 2)
    idx = xp.reshape(_NW, chunks, _CSZ).astype(jnp.int32)
    xe = _sc_gather(idx, table)
    x128 = xe.reshape(n * d // 128, 128)
    w50 = W_proj.reshape(nt, upack * d, nc)
    return _tc_head(x128, w50, b_proj.reshape(1, nc), bsz, nt)
